# baseline (speedup 1.0000x reference)
import jax
import jax.numpy as jnp
from jax import lax
from jax.experimental import pallas as pl
from jax.experimental.pallas import tpu as pltpu
from jax.experimental.pallas import tpu_sc as plsc

_NC = 2
_NS = 16
_NW = _NC * _NS
_LANES = 16
_OUT_DIM = 1024
_GPR = _OUT_DIM // _LANES
_CHUNK = 16
_CROWS = _CHUNK * _GPR
_NBUF = 3
_TOTAL_ROWS = 4 * 4096
_ROWS_PER_W = _TOTAL_ROWS // _NW
_NCHUNKS = _ROWS_PER_W // _CHUNK
_TBL_ROWS = 4096


def _sc_body(x_hbm, t_hbm, o_hbm, xbuf, tbuf, in_sems, t_sems, out_sems):
    cid = lax.axis_index("c")
    sid = lax.axis_index("s")
    wid = sid * _NC + cid
    base = wid * _ROWS_PER_W * _GPR
    tbase = (wid * _ROWS_PER_W % _TBL_ROWS) * _GPR

    def in_copy(k):
        slot = k % _NBUF
        return pltpu.make_async_copy(
            x_hbm.at[pl.ds(base + k * _CROWS, _CROWS), :],
            xbuf.at[slot],
            in_sems.at[slot],
        )

    def t_copy(k):
        slot = k % _NBUF
        return pltpu.make_async_copy(
            t_hbm.at[pl.ds(tbase + k * _CROWS, _CROWS), :],
            tbuf.at[slot],
            t_sems.at[slot],
        )

    def out_copy(k):
        slot = k % _NBUF
        return pltpu.make_async_copy(
            xbuf.at[slot],
            o_hbm.at[pl.ds(base + k * _CROWS, _CROWS), :],
            out_sems.at[slot],
        )

    for k in range(_NBUF):
        in_copy(k).start()
        t_copy(k).start()

    for k in range(_NCHUNKS):
        slot = k % _NBUF
        in_copy(k).wait()
        t_copy(k).wait()

        def body(j, _):
            xbuf[slot, j] = xbuf[slot, j] + tbuf[slot, j]
            return 0

        lax.fori_loop(0, _CROWS, body, 0, unroll=32)
        out_copy(k).start()
        nxt = k + _NBUF
        if nxt < _NCHUNKS:
            out_copy(k).wait()
            in_copy(nxt).start()
            t_copy(nxt).start()

    for k in range(_NCHUNKS - _NBUF, _NCHUNKS):
        out_copy(k).wait()


def kernel(inputs, pos_table):
    batch, seq_len, out_dim = inputs.shape
    flat = inputs.reshape(batch * seq_len * _GPR, _LANES)
    tbl = pos_table.reshape(seq_len * _GPR, _LANES)
    mesh = plsc.VectorSubcoreMesh(core_axis_name="c", subcore_axis_name="s")
    out = pl.kernel(
        _sc_body,
        out_type=jax.ShapeDtypeStruct(flat.shape, flat.dtype),
        mesh=mesh,
        compiler_params=pltpu.CompilerParams(use_tc_tiling_on_sc=False),
        scratch_types=[
            pltpu.VMEM((_NBUF, _CROWS, _LANES), jnp.float32),
            pltpu.VMEM((_NBUF, _CROWS, _LANES), jnp.float32),
            pltpu.SemaphoreType.DMA((_NBUF,)),
            pltpu.SemaphoreType.DMA((_NBUF,)),
            pltpu.SemaphoreType.DMA((_NBUF,)),
        ],
    )(flat, tbl)
    return out.reshape(batch, seq_len, out_dim)


# speedup vs baseline: 1.7897x; 1.7897x over previous
import jax
import jax.numpy as jnp
from jax import lax
from jax.experimental import pallas as pl
from jax.experimental.pallas import tpu as pltpu
from jax.experimental.pallas import tpu_sc as plsc

_LANES = 16
_OUT_DIM = 1024
_GPR = _OUT_DIM // _LANES
_TOTAL_ROWS = 4 * 4096
_TBL_ROWS = 4096

_R_SC = 2048
_NW = 32
_SC_RPW = _R_SC // _NW
_SC_CHUNK = 16
_SC_CROWS = _SC_CHUNK * _GPR
_SC_NCHUNKS = _SC_RPW // _SC_CHUNK
_SC_NBUF = 3

_TC_CHUNK = 1024
_TC_DEPTH = 4
_TC_ROW0 = _R_SC
_TC_NCHUNKS = (_TOTAL_ROWS - _R_SC) // _TC_CHUNK


def _sc_body(x_hbm, t_hbm, o_hbm, xbuf, tbuf, in_sems, t_sems, out_sems):
    cid = lax.axis_index("c")
    sid = lax.axis_index("s")
    wid = sid * 2 + cid
    base = wid * _SC_RPW * _GPR
    tbase = (wid * _SC_RPW % _TBL_ROWS) * _GPR

    def in_copy(k):
        slot = k % _SC_NBUF
        return pltpu.make_async_copy(
            x_hbm.at[pl.ds(base + k * _SC_CROWS, _SC_CROWS), :],
            xbuf.at[slot], in_sems.at[slot])

    def t_copy(k):
        slot = k % _SC_NBUF
        return pltpu.make_async_copy(
            t_hbm.at[pl.ds(tbase + k * _SC_CROWS, _SC_CROWS), :],
            tbuf.at[slot], t_sems.at[slot])

    def out_copy(k):
        slot = k % _SC_NBUF
        return pltpu.make_async_copy(
            xbuf.at[slot],
            o_hbm.at[pl.ds(base + k * _SC_CROWS, _SC_CROWS), :],
            out_sems.at[slot])

    for k in range(_SC_NBUF):
        in_copy(k).start()
        t_copy(k).start()

    for k in range(_SC_NCHUNKS):
        slot = k % _SC_NBUF
        in_copy(k).wait()
        t_copy(k).wait()

        def body(j, _):
            xbuf[slot, j] = xbuf[slot, j] + tbuf[slot, j]
            return 0

        lax.fori_loop(0, _SC_CROWS, body, 0, unroll=32)
        out_copy(k).start()
        nxt = k + _SC_NBUF
        if nxt < _SC_NCHUNKS:
            out_copy(k).wait()
            in_copy(nxt).start()
            t_copy(nxt).start()

    for k in range(max(0, _SC_NCHUNKS - _SC_NBUF), _SC_NCHUNKS):
        out_copy(k).wait()


def _tc_body(x_hbm, t_hbm, o_hbm, in_buf, tbl, out_buf, in_sems, out_sems, tbl_sem):
    tbl_copy = pltpu.make_async_copy(t_hbm, tbl, tbl_sem)
    tbl_copy.start()

    def in_copy(c):
        slot = c % _TC_DEPTH
        return pltpu.make_async_copy(
            x_hbm.at[pl.ds(_TC_ROW0 + c * _TC_CHUNK, _TC_CHUNK), :],
            in_buf.at[slot], in_sems.at[slot])

    def out_copy(c):
        slot = c % _TC_DEPTH
        return pltpu.make_async_copy(
            out_buf.at[slot],
            o_hbm.at[pl.ds(_TC_ROW0 + c * _TC_CHUNK, _TC_CHUNK), :],
            out_sems.at[slot])

    for c in range(_TC_DEPTH):
        in_copy(c).start()
    tbl_copy.wait()

    for c in range(_TC_NCHUNKS):
        slot = c % _TC_DEPTH
        in_copy(c).wait()
        if c >= _TC_DEPTH:
            out_copy(c - _TC_DEPTH).wait()
        off = ((_TC_ROW0 + c * _TC_CHUNK) % _TBL_ROWS)
        out_buf[slot] = in_buf[slot] + tbl[pl.ds(off, _TC_CHUNK), :]
        out_copy(c).start()
        nxt = c + _TC_DEPTH
        if nxt < _TC_NCHUNKS:
            in_copy(nxt).start()

    for c in range(_TC_NCHUNKS - _TC_DEPTH, _TC_NCHUNKS):
        out_copy(c).wait()


def kernel(inputs, pos_table):
    batch, seq_len, out_dim = inputs.shape
    flat = inputs.reshape(batch * seq_len, out_dim)
    flat16 = inputs.reshape(batch * seq_len * _GPR, _LANES)
    tbl16 = pos_table.reshape(seq_len * _GPR, _LANES)

    sc_out = pl.kernel(
        _sc_body,
        out_type=jax.ShapeDtypeStruct((_R_SC * _GPR, _LANES), jnp.float32),
        mesh=plsc.VectorSubcoreMesh(core_axis_name="c", subcore_axis_name="s"),
        compiler_params=pltpu.CompilerParams(use_tc_tiling_on_sc=False),
        scratch_types=[
            pltpu.VMEM((_SC_NBUF, _SC_CROWS, _LANES), jnp.float32),
            pltpu.VMEM((_SC_NBUF, _SC_CROWS, _LANES), jnp.float32),
            pltpu.SemaphoreType.DMA((_SC_NBUF,)),
            pltpu.SemaphoreType.DMA((_SC_NBUF,)),
            pltpu.SemaphoreType.DMA((_SC_NBUF,)),
        ],
    )(flat16, tbl16)

    tc_out = pl.pallas_call(
        _tc_body,
        in_specs=[
            pl.BlockSpec(memory_space=pltpu.MemorySpace.HBM),
            pl.BlockSpec(memory_space=pltpu.MemorySpace.HBM),
        ],
        out_specs=pl.BlockSpec(memory_space=pltpu.MemorySpace.HBM),
        out_shape=jax.ShapeDtypeStruct(flat.shape, flat.dtype),
        scratch_shapes=[
            pltpu.VMEM((_TC_DEPTH, _TC_CHUNK, out_dim), jnp.float32),
            pltpu.VMEM((seq_len, out_dim), jnp.float32),
            pltpu.VMEM((_TC_DEPTH, _TC_CHUNK, out_dim), jnp.float32),
            pltpu.SemaphoreType.DMA((_TC_DEPTH,)),
            pltpu.SemaphoreType.DMA((_TC_DEPTH,)),
            pltpu.SemaphoreType.DMA,
        ],
    )(flat, pos_table)

    merged = lax.dynamic_update_slice(
        tc_out, sc_out.reshape(_R_SC, _OUT_DIM), (0, 0))
    return merged.reshape(batch, seq_len, out_dim)


# speedup vs baseline: 5.8003x; 3.2410x over previous
import jax
import jax.numpy as jnp
from jax.experimental import pallas as pl
from jax.experimental.pallas import tpu as pltpu

_CHUNK = 1024
_DEPTH = 5
_N_CHUNKS = 16
_TBL_ROWS = 4096


def _body(x_hbm, t_hbm, o_hbm, in_buf, tbl, out_buf, in_sems, out_sems, tbl_sem):
    tbl_copy = pltpu.make_async_copy(t_hbm, tbl, tbl_sem)
    tbl_copy.start()

    def in_copy(c):
        slot = c % _DEPTH
        return pltpu.make_async_copy(
            x_hbm.at[pl.ds(c * _CHUNK, _CHUNK), :],
            in_buf.at[slot],
            in_sems.at[slot],
        )

    def out_copy(c):
        slot = c % _DEPTH
        return pltpu.make_async_copy(
            out_buf.at[slot],
            o_hbm.at[pl.ds(c * _CHUNK, _CHUNK), :],
            out_sems.at[slot],
        )

    for c in range(_DEPTH):
        in_copy(c).start()
    tbl_copy.wait()

    for c in range(_N_CHUNKS):
        slot = c % _DEPTH
        in_copy(c).wait()
        if c >= _DEPTH:
            out_copy(c - _DEPTH).wait()
        off = (c % (_TBL_ROWS // _CHUNK)) * _CHUNK
        out_buf[slot] = in_buf[slot] + tbl[pl.ds(off, _CHUNK), :]
        out_copy(c).start()
        nxt = c + _DEPTH
        if nxt < _N_CHUNKS:
            in_copy(nxt).start()

    for c in range(_N_CHUNKS - _DEPTH, _N_CHUNKS):
        out_copy(c).wait()


def kernel(inputs, pos_table):
    batch, seq_len, out_dim = inputs.shape
    flat = inputs.reshape(batch * seq_len, out_dim)
    out = pl.pallas_call(
        _body,
        in_specs=[
            pl.BlockSpec(memory_space=pltpu.MemorySpace.HBM),
            pl.BlockSpec(memory_space=pltpu.MemorySpace.HBM),
        ],
        out_specs=pl.BlockSpec(memory_space=pltpu.MemorySpace.HBM),
        out_shape=jax.ShapeDtypeStruct(flat.shape, flat.dtype),
        scratch_shapes=[
            pltpu.VMEM((_DEPTH, _CHUNK, out_dim), jnp.float32),
            pltpu.VMEM((seq_len, out_dim), jnp.float32),
            pltpu.VMEM((_DEPTH, _CHUNK, out_dim), jnp.float32),
            pltpu.SemaphoreType.DMA((_DEPTH,)),
            pltpu.SemaphoreType.DMA((_DEPTH,)),
            pltpu.SemaphoreType.DMA,
        ],
    )(flat, pos_table)
    return out.reshape(batch, seq_len, out_dim)
